# baseline (speedup 1.0000x reference)
import jax
import jax.numpy as jnp
from jax import lax
from jax.experimental import pallas as pl
from jax.experimental.pallas import tpu as pltpu
from jax.experimental.pallas import tpu_sc as plsc

D = 128
SCALE = float(D) ** 0.5
NW = 32
SEQ = 200
CHUNK = 50
LANES = 16
SLOTS = 8


def _maybe(cond, fn):
    if isinstance(cond, bool):
        if cond:
            fn()
    else:
        pl.when(cond)(fn)


def _chunk(j, p, m, table_hbm, out_hbm, idx_v, pos_v, bufs, gs, ss, gbase):
    s = p
    pltpu.make_async_copy(table_hbm.at[idx_v.at[j]], bufs[s], gs[s]).wait()

    def _scatter_jm2():
        pltpu.async_copy(
            bufs[(p - 2) % SLOTS],
            out_hbm.at[pl.ds(gbase + (j - 2) * CHUNK, CHUNK)],
            ss[(p - 2) % SLOTS])
    _maybe(j >= 2, _scatter_jm2)

    def _wait_old_scatter():
        c = j + 3 - SLOTS
        pltpu.make_async_copy(
            bufs[(p + 3) % SLOTS],
            out_hbm.at[pl.ds(gbase + c * CHUNK, CHUNK)],
            ss[(p + 3) % SLOTS]).wait()
    _maybe(j + 3 - SLOTS >= 0, _wait_old_scatter)

    def _next_gather():
        pltpu.async_copy(table_hbm.at[idx_v.at[j + 3]],
                         bufs[(p + 3) % SLOTS], gs[(p + 3) % SLOTS])
    _maybe(j + 3 < m, _next_gather)

    poff = (p * CHUNK) % SEQ
    buf = bufs[s]

    @plsc.parallel_loop(0, CHUNK, step=1, unroll=4)
    def _compute(r):
        for c in range(D // LANES):
            sl = pl.ds(c * LANES, LANES)
            buf[r, sl] = buf[r, sl] * SCALE + pos_v[poff + r, sl]


def _body(x_hbm, table_hbm, pos_hbm, out_hbm, idx_v, pos_v, refs):
    bufs = refs[:SLOTS]
    gs = refs[SLOTS:2 * SLOTS]
    ss = refs[2 * SLOTS:]
    m = idx_v.shape[0]
    wid = lax.axis_index("s") * 2 + lax.axis_index("c")
    gbase = wid * (m * CHUNK)

    pltpu.sync_copy(x_hbm.at[pl.ds(wid * m, m)], idx_v)
    pltpu.sync_copy(pos_hbm, pos_v)

    for b in range(3):
        pltpu.async_copy(table_hbm.at[idx_v.at[b]], bufs[b], gs[b])

    def iter_body(t, carry):
        for p in range(SLOTS):
            _chunk(SLOTS * t + p, p, m, table_hbm, out_hbm, idx_v, pos_v,
                   bufs, gs, ss, gbase)
        return carry

    lax.fori_loop(0, m // SLOTS, iter_body, 0)

    def _scatter(c):
        pltpu.async_copy(
            bufs[c % SLOTS], out_hbm.at[pl.ds(gbase + c * CHUNK, CHUNK)],
            ss[c % SLOTS])

    def _wait(c):
        pltpu.make_async_copy(
            bufs[c % SLOTS], out_hbm.at[pl.ds(gbase + c * CHUNK, CHUNK)],
            ss[c % SLOTS]).wait()

    _scatter(m - 2)
    for c in range(m + 3 - SLOTS, m - 2):
        _wait(c)
    _scatter(m - 1)
    _wait(m - 2)
    _wait(m - 1)


def kernel(x, table, pos_encoding):
    B, S = x.shape
    N = B * S
    n_chunks = N // CHUNK
    x2 = x.reshape(n_chunks, CHUNK)
    seq = pos_encoding.shape[0]

    mesh = plsc.VectorSubcoreMesh(core_axis_name="c", subcore_axis_name="s")

    def body(x_hbm, table_hbm, pos_hbm, out_hbm, idx_v, pos_v, *refs):
        _body(x_hbm, table_hbm, pos_hbm, out_hbm, idx_v, pos_v, refs)

    run = pl.kernel(
        body,
        out_type=jax.ShapeDtypeStruct((N, D), jnp.float32),
        mesh=mesh,
        compiler_params=pltpu.CompilerParams(use_tc_tiling_on_sc=False),
        scratch_types=(
            [pltpu.VMEM((n_chunks // NW, CHUNK), jnp.int32),
             pltpu.VMEM((seq, D), jnp.float32)]
            + [pltpu.VMEM((CHUNK, D), jnp.float32)
               for _ in range(SLOTS)]
            + [pltpu.SemaphoreType.DMA for _ in range(2 * SLOTS)]
        ),
    )
    out = run(x2, table, pos_encoding)
    return out.reshape(B, S, D)
